# baseline (speedup 1.0000x reference)
import functools

import jax
import jax.numpy as jnp
from jax import lax
from jax.experimental import pallas as pl
from jax.experimental.pallas import tpu as pltpu
from jax.experimental.pallas import tpu_sc as plsc

K_NN = 16
L = 4
EPS = 1e-5


def _topk_body(xyzr_ref, xyzc_ref, knn_ref):
    xr = xyzr_ref[0, :, 0:1]
    yr = xyzr_ref[0, :, 1:2]
    zr = xyzr_ref[0, :, 2:3]
    xc = xyzc_ref[0, 0:1, :]
    yc = xyzc_ref[0, 1:2, :]
    zc = xyzc_ref[0, 2:3, :]
    sqr = xr * xr + yr * yr + zr * zr
    sqc = xc * xc + yc * yc + zc * zc
    bf = lambda t: t.astype(jnp.bfloat16).astype(jnp.float32)
    dot = bf(xr) * bf(xc) + bf(yr) * bf(yc) + bf(zr) * bf(zc)
    arr = (sqr + sqc) - 2.0 * dot
    R = arr.shape[0]
    Nn = arr.shape[1]
    fcol = lax.broadcasted_iota(jnp.int32, (R, Nn), 1).astype(jnp.float32)
    big_f = jnp.float32(1e9)
    inf = jnp.float32(jnp.inf)
    outs = []
    for _ in range(K_NN):
        m = jnp.min(arr, axis=1, keepdims=True)
        cand = jnp.where(arr == m, fcol, big_f)
        amin = jnp.min(cand, axis=1, keepdims=True)
        outs.append(amin)
        arr = jnp.where(fcol == amin, inf, arr)
    knn_ref[0, :, :] = jnp.concatenate(outs, axis=1).astype(jnp.int32)


def _knn_topk(xyz, B, N):
    R = 256
    xyzT = jnp.transpose(xyz, (0, 2, 1))
    return pl.pallas_call(
        _topk_body,
        grid=(B, N // R),
        in_specs=[
            pl.BlockSpec((1, R, 3), lambda b, t: (b, t, 0)),
            pl.BlockSpec((1, 3, N), lambda b, t: (b, 0, 0)),
        ],
        out_specs=pl.BlockSpec((1, R, K_NN), lambda b, t: (b, t, 0)),
        out_shape=jax.ShapeDtypeStruct((B, N, K_NN), jnp.int32),
    )(xyz, xyzT)


def _proj_body(feat_ref, xyz_ref, w1_ref, p_ref, hx_ref):
    w1f = w1_ref[:, 0:128]
    fb = feat_ref[0]
    pf = lax.dot_general(fb, w1f, (((0,), (1,)), ((), ())),
                         preferred_element_type=jnp.float32)
    xb = xyz_ref[0]
    hx = (xb[:, 0:1] * w1_ref[:, 128:129].T
          + xb[:, 1:2] * w1_ref[:, 129:130].T
          + xb[:, 2:3] * w1_ref[:, 130:131].T)
    hx_ref[0] = hx
    p_ref[0] = pf + hx


def _projections(feat, xyz, W1, B, N, D):
    Rn = 1024
    return pl.pallas_call(
        _proj_body,
        grid=(B, N // Rn),
        in_specs=[
            pl.BlockSpec((1, D, Rn), lambda b, t: (b, 0, t)),
            pl.BlockSpec((1, Rn, 3), lambda b, t: (b, t, 0)),
            pl.BlockSpec((D, D + 3), lambda b, t: (0, 0)),
        ],
        out_specs=[
            pl.BlockSpec((1, Rn, D), lambda b, t: (b, t, 0)),
            pl.BlockSpec((1, Rn, D), lambda b, t: (b, t, 0)),
        ],
        out_shape=[
            jax.ShapeDtypeStruct((B, N, D), jnp.float32),
            jax.ShapeDtypeStruct((B, N, D), jnp.float32),
        ],
    )(feat, xyz, W1)


def _sc_walk_gather(knn2, rand_steps, pflat, B, N, D):
    info = plsc.get_sparse_core_info()
    NC, NS = info.num_cores, info.num_subcores
    NW = NC * NS
    C = N // NW
    mesh = plsc.VectorSubcoreMesh(core_axis_name="c", subcore_axis_name="s")

    @functools.partial(
        pl.kernel,
        mesh=mesh,
        compiler_params=pltpu.CompilerParams(
            needs_layout_passes=False, use_tc_tiling_on_sc=False),
        out_type=jax.ShapeDtypeStruct((B * L * N, D), jnp.float32),
        scratch_types=[
            pltpu.VMEM((C, K_NN), jnp.int32),
            pltpu.VMEM((C,), jnp.int32),
            pltpu.VMEM(((L - 1) * C,), jnp.int32),
            pltpu.VMEM((C, D), jnp.float32),
            pltpu.SemaphoreType.DMA,
            pltpu.SemaphoreType.DMA,
        ],
    )
    def k(knn_hbm, rand_hbm, p_hbm, out_hbm, krows_v, cur_v, rnd_v,
          rows_v, semp, semk):
        wid = lax.axis_index("s") * NC + lax.axis_index("c")
        base = wid * C
        for b in range(B):
            for s in range(L - 1):
                pltpu.sync_copy(rand_hbm.at[s, b, pl.ds(base, C)],
                                rnd_v.at[pl.ds(s * C, C)])
            for j in range(C // 16):
                cur_v[pl.ds(j * 16, 16)] = (
                    lax.iota(jnp.int32, 16) + (b * N + base + j * 16))
            knn_dma = pltpu.async_copy(knn_hbm.at[cur_v], krows_v, semk)
            pltpu.sync_copy(p_hbm.at[pl.ds(b * N + base, C)], rows_v)
            pltpu.sync_copy(rows_v, out_hbm.at[pl.ds(b * L * N + base, C)])
            for s in range(L - 1):
                knn_dma.wait()
                for j in range(C // 16):
                    loc = lax.iota(jnp.int32, 16) + (j * 16)
                    rd = rnd_v[pl.ds(s * C + j * 16, 16)]
                    nxt = plsc.load_gather(krows_v, [loc, rd])
                    cur_v[pl.ds(j * 16, 16)] = nxt + (b * N)
                p_dma = pltpu.async_copy(p_hbm.at[cur_v], rows_v, semp)
                if s < L - 2:
                    knn_dma = pltpu.async_copy(
                        knn_hbm.at[cur_v], krows_v, semk)
                p_dma.wait()
                pltpu.sync_copy(
                    rows_v,
                    out_hbm.at[pl.ds((b * L + s + 1) * N + base, C)])

    return k(knn2, rand_steps, pflat)


def _mvinv(stats_row, count, g, bparm):
    m = stats_row[0:1, :] / count
    v = stats_row[1:2, :] / count - m * m
    inv = lax.rsqrt(v + EPS)
    scale = inv * g.reshape(1, -1)
    shift = bparm.reshape(1, -1) - m * scale
    return scale, shift


def _conv2_body(g_ref, hx_ref, w2_ref, g1_ref, b1_ref,
                m2_ref, s2_ref, s1_scr, *, count1):
    p = pl.program_id(0)
    b = pl.program_id(1)
    t = pl.program_id(2)
    li = pl.program_id(3)
    first = (b == 0) & (t == 0) & (li == 0)

    h = g_ref[0, 0] - hx_ref[0, 0]

    @pl.when((p == 0) & first)
    def _():
        s1_scr[...] = jnp.zeros_like(s1_scr)
        s2_ref[...] = jnp.zeros_like(s2_ref)

    @pl.when(p == 0)
    def _():
        s1_scr[0:1, :] += jnp.sum(h, axis=0, keepdims=True)
        s1_scr[1:2, :] += jnp.sum(h * h, axis=0, keepdims=True)

    @pl.when(p == 1)
    def _():
        scale, shift = _mvinv(s1_scr, count1, g1_ref[...], b1_ref[...])
        a = jnp.maximum(h * scale + shift, 0.0)
        h2 = lax.dot_general(a, w2_ref[...], (((1,), (1,)), ((), ())),
                             preferred_element_type=jnp.float32)
        s2_ref[0:1, :] += jnp.sum(h2, axis=0, keepdims=True)
        s2_ref[1:2, :] += jnp.sum(h2 * h2, axis=0, keepdims=True)

        @pl.when(li == 0)
        def _():
            m2_ref[0] = h2

        @pl.when(li > 0)
        def _():
            m2_ref[0] = jnp.maximum(m2_ref[0], h2)


def _conv2(gall, hxt, W2, g1, b1, B, N, D, Rn=2048):
    count1 = float(B * N * L)
    return pl.pallas_call(
        functools.partial(_conv2_body, count1=count1),
        grid=(2, B, N // Rn, L),
        in_specs=[
            pl.BlockSpec((1, 1, Rn, D), lambda p, b, t, li: (b, li, t, 0)),
            pl.BlockSpec((1, 1, Rn, D), lambda p, b, t, li: (b, 0, t, 0)),
            pl.BlockSpec((D, D), lambda p, b, t, li: (0, 0)),
            pl.BlockSpec((D,), lambda p, b, t, li: (0,)),
            pl.BlockSpec((D,), lambda p, b, t, li: (0,)),
        ],
        out_specs=[
            pl.BlockSpec((1, Rn, D), lambda p, b, t, li: (b * p, t * p, 0)),
            pl.BlockSpec((2, D), lambda p, b, t, li: (0, 0)),
        ],
        out_shape=[
            jax.ShapeDtypeStruct((B, N, D), jnp.float32),
            jax.ShapeDtypeStruct((2, D), jnp.float32),
        ],
        scratch_shapes=[pltpu.VMEM((2, D), jnp.float32)],
    )(gall, hxt, W2, g1, b1)


def _resid_body(m2_ref, featT_ref, s2_ref, g2_ref, b2_ref, u_ref, su_ref,
                *, count2):
    b = pl.program_id(0)
    t = pl.program_id(1)

    @pl.when((b == 0) & (t == 0))
    def _():
        su_ref[...] = jnp.zeros_like(su_ref)

    scale, shift = _mvinv(s2_ref, count2, g2_ref[...], b2_ref[...])
    agg = jnp.maximum(m2_ref[0] * scale + shift, 0.0)
    u = featT_ref[0] + agg
    u_ref[0] = u
    su_ref[0:1, :] += jnp.sum(u, axis=0, keepdims=True)
    su_ref[1:2, :] += jnp.sum(u * u, axis=0, keepdims=True)


def _resid(m2, featT, stats2, g2, b2, B, N, D, Rn=2048):
    count2 = float(B * N * L)
    return pl.pallas_call(
        functools.partial(_resid_body, count2=count2),
        grid=(B, N // Rn),
        in_specs=[
            pl.BlockSpec((1, Rn, D), lambda b, t: (b, t, 0)),
            pl.BlockSpec((1, Rn, D), lambda b, t: (b, t, 0)),
            pl.BlockSpec((2, D), lambda b, t: (0, 0)),
            pl.BlockSpec((D,), lambda b, t: (0,)),
            pl.BlockSpec((D,), lambda b, t: (0,)),
        ],
        out_specs=[
            pl.BlockSpec((1, Rn, D), lambda b, t: (b, t, 0)),
            pl.BlockSpec((2, D), lambda b, t: (0, 0)),
        ],
        out_shape=[
            jax.ShapeDtypeStruct((B, N, D), jnp.float32),
            jax.ShapeDtypeStruct((2, D), jnp.float32),
        ],
    )(m2, featT, stats2, g2, b2)


def _mlp1_body(u_ref, su_ref, w3_ref, gn_ref, bn_ref, ff_ref, s3_ref,
               *, countu):
    b = pl.program_id(0)
    t = pl.program_id(1)

    @pl.when((b == 0) & (t == 0))
    def _():
        s3_ref[...] = jnp.zeros_like(s3_ref)

    scale, shift = _mvinv(su_ref, countu, gn_ref[...], bn_ref[...])
    f = u_ref[0] * scale + shift
    ff = lax.dot_general(f, w3_ref[...], (((1,), (1,)), ((), ())),
                         preferred_element_type=jnp.float32)
    ff_ref[0] = ff
    s3_ref[0:1, :] += jnp.sum(ff, axis=0, keepdims=True)
    s3_ref[1:2, :] += jnp.sum(ff * ff, axis=0, keepdims=True)


def _mlp1(u, statsu, W3, gn, bn_, B, N, D, Rn=2048):
    countu = float(B * N)
    return pl.pallas_call(
        functools.partial(_mlp1_body, countu=countu),
        grid=(B, N // Rn),
        in_specs=[
            pl.BlockSpec((1, Rn, D), lambda b, t: (b, t, 0)),
            pl.BlockSpec((2, D), lambda b, t: (0, 0)),
            pl.BlockSpec((2 * D, D), lambda b, t: (0, 0)),
            pl.BlockSpec((D,), lambda b, t: (0,)),
            pl.BlockSpec((D,), lambda b, t: (0,)),
        ],
        out_specs=[
            pl.BlockSpec((1, Rn, 2 * D), lambda b, t: (b, t, 0)),
            pl.BlockSpec((2, 2 * D), lambda b, t: (0, 0)),
        ],
        out_shape=[
            jax.ShapeDtypeStruct((B, N, 2 * D), jnp.float32),
            jax.ShapeDtypeStruct((2, 2 * D), jnp.float32),
        ],
    )(u, statsu, W3, gn, bn_)


def _mlp2_body(ff_ref, s3_ref, u_ref, su_ref, w4_ref, g3_ref, b3_ref,
               gn_ref, bn_ref, v_ref, sv_ref, *, countu):
    b = pl.program_id(0)
    t = pl.program_id(1)

    @pl.when((b == 0) & (t == 0))
    def _():
        sv_ref[...] = jnp.zeros_like(sv_ref)

    scale3, shift3 = _mvinv(s3_ref, countu, g3_ref[...], b3_ref[...])
    a = jnp.maximum(ff_ref[0] * scale3 + shift3, 0.0)
    ff2 = lax.dot_general(a, w4_ref[...], (((1,), (1,)), ((), ())),
                          preferred_element_type=jnp.float32)
    scaleu, shiftu = _mvinv(su_ref, countu, gn_ref[...], bn_ref[...])
    f = u_ref[0] * scaleu + shiftu
    v = f + ff2
    v_ref[0] = v
    sv_ref[0:1, :] += jnp.sum(v, axis=0, keepdims=True)
    sv_ref[1:2, :] += jnp.sum(v * v, axis=0, keepdims=True)


def _mlp2(ff, stats3, u, statsu, W4, g3, b3, gn, bn_, B, N, D, Rn=2048):
    countu = float(B * N)
    return pl.pallas_call(
        functools.partial(_mlp2_body, countu=countu),
        grid=(B, N // Rn),
        in_specs=[
            pl.BlockSpec((1, Rn, 2 * D), lambda b, t: (b, t, 0)),
            pl.BlockSpec((2, 2 * D), lambda b, t: (0, 0)),
            pl.BlockSpec((1, Rn, D), lambda b, t: (b, t, 0)),
            pl.BlockSpec((2, D), lambda b, t: (0, 0)),
            pl.BlockSpec((D, 2 * D), lambda b, t: (0, 0)),
            pl.BlockSpec((2 * D,), lambda b, t: (0,)),
            pl.BlockSpec((2 * D,), lambda b, t: (0,)),
            pl.BlockSpec((D,), lambda b, t: (0,)),
            pl.BlockSpec((D,), lambda b, t: (0,)),
        ],
        out_specs=[
            pl.BlockSpec((1, Rn, D), lambda b, t: (b, t, 0)),
            pl.BlockSpec((2, D), lambda b, t: (0, 0)),
        ],
        out_shape=[
            jax.ShapeDtypeStruct((B, N, D), jnp.float32),
            jax.ShapeDtypeStruct((2, D), jnp.float32),
        ],
    )(ff, stats3, u, statsu, W4, g3, b3, gn, bn_)


def _final_body(v_ref, sv_ref, gn2_ref, bn2_ref, o_ref, *, countu):
    scale, shift = _mvinv(sv_ref, countu, gn2_ref[...], bn2_ref[...])
    o_ref[0] = v_ref[0] * scale + shift


def _final(v, statsv, gn2, bn2, B, N, D, Rn=2048):
    countu = float(B * N)
    return pl.pallas_call(
        functools.partial(_final_body, countu=countu),
        grid=(B, N // Rn),
        in_specs=[
            pl.BlockSpec((1, Rn, D), lambda b, t: (b, t, 0)),
            pl.BlockSpec((2, D), lambda b, t: (0, 0)),
            pl.BlockSpec((D,), lambda b, t: (0,)),
            pl.BlockSpec((D,), lambda b, t: (0,)),
        ],
        out_specs=pl.BlockSpec((1, Rn, D), lambda b, t: (b, t, 0)),
        out_shape=jax.ShapeDtypeStruct((B, N, D), jnp.float32),
    )(v, statsv, gn2, bn2)


def kernel(xyz, feat, rand_steps, W1, g1, b1, W2, g2, b2, gn, bn_, W3, g3,
           b3, W4, gn2, bn2):
    B, N, _ = xyz.shape
    D = feat.shape[1]

    knn = _knn_topk(xyz, B, N)
    p, hxt = _projections(feat, xyz, W1, B, N, D)

    gall = _sc_walk_gather(
        knn.reshape(B * N, K_NN), rand_steps, p.reshape(B * N, D), B, N, D)
    gall = gall.reshape(B, L, N, D)
    hxt4 = hxt.reshape(B, 1, N, D)

    m2, stats2 = _conv2(gall, hxt4, W2, g1, b1, B, N, D)

    featT = jnp.transpose(feat, (0, 2, 1))
    u, statsu = _resid(m2, featT, stats2, g2, b2, B, N, D)
    ff, stats3 = _mlp1(u, statsu, W3, gn, bn_, B, N, D)
    v, statsv = _mlp2(ff, stats3, u, statsu, W4, g3, b3, gn, bn_, B, N, D)
    out = _final(v, statsv, gn2, bn2, B, N, D)
    return jnp.transpose(out, (0, 2, 1))


# speedup vs baseline: 1.0041x; 1.0041x over previous
import functools

import jax
import jax.numpy as jnp
from jax import lax
from jax.experimental import pallas as pl
from jax.experimental.pallas import tpu as pltpu
from jax.experimental.pallas import tpu_sc as plsc

K_NN = 16
L = 4
EPS = 1e-5


def _topk_body(xyzr_ref, xyzc_ref, knn_ref):
    xr = xyzr_ref[0, :, 0:1]
    yr = xyzr_ref[0, :, 1:2]
    zr = xyzr_ref[0, :, 2:3]
    xc = xyzc_ref[0, 0:1, :]
    yc = xyzc_ref[0, 1:2, :]
    zc = xyzc_ref[0, 2:3, :]
    sqr = xr * xr + yr * yr + zr * zr
    sqc = xc * xc + yc * yc + zc * zc
    bf = lambda t: t.astype(jnp.bfloat16).astype(jnp.float32)
    dot = bf(xr) * bf(xc) + bf(yr) * bf(yc) + bf(zr) * bf(zc)
    arr = (sqr + sqc) - 2.0 * dot
    R = arr.shape[0]
    Nn = arr.shape[1]
    fcol = lax.broadcasted_iota(jnp.int32, (R, Nn), 1).astype(jnp.float32)
    big_f = jnp.float32(1e9)
    inf = jnp.float32(jnp.inf)
    outs = []
    for _ in range(K_NN):
        m = jnp.min(arr, axis=1, keepdims=True)
        cand = jnp.where(arr == m, fcol, big_f)
        amin = jnp.min(cand, axis=1, keepdims=True)
        outs.append(amin)
        arr = jnp.where(fcol == amin, inf, arr)
    knn_ref[0, :, :] = jnp.concatenate(outs, axis=1).astype(jnp.int32)


def _knn_topk(xyz, B, N):
    R = 512
    xyzT = jnp.transpose(xyz, (0, 2, 1))
    return pl.pallas_call(
        _topk_body,
        grid=(B, N // R),
        in_specs=[
            pl.BlockSpec((1, R, 3), lambda b, t: (b, t, 0)),
            pl.BlockSpec((1, 3, N), lambda b, t: (b, 0, 0)),
        ],
        out_specs=pl.BlockSpec((1, R, K_NN), lambda b, t: (b, t, 0)),
        out_shape=jax.ShapeDtypeStruct((B, N, K_NN), jnp.int32),
    )(xyz, xyzT)


def _proj_body(feat_ref, xyz_ref, w1_ref, p_ref, hx_ref):
    w1f = w1_ref[:, 0:128]
    fb = feat_ref[0]
    pf = lax.dot_general(fb, w1f, (((0,), (1,)), ((), ())),
                         preferred_element_type=jnp.float32)
    xb = xyz_ref[0]
    hx = (xb[:, 0:1] * w1_ref[:, 128:129].T
          + xb[:, 1:2] * w1_ref[:, 129:130].T
          + xb[:, 2:3] * w1_ref[:, 130:131].T)
    hx_ref[0] = hx
    p_ref[0] = pf + hx


def _projections(feat, xyz, W1, B, N, D):
    Rn = 1024
    return pl.pallas_call(
        _proj_body,
        grid=(B, N // Rn),
        in_specs=[
            pl.BlockSpec((1, D, Rn), lambda b, t: (b, 0, t)),
            pl.BlockSpec((1, Rn, 3), lambda b, t: (b, t, 0)),
            pl.BlockSpec((D, D + 3), lambda b, t: (0, 0)),
        ],
        out_specs=[
            pl.BlockSpec((1, Rn, D), lambda b, t: (b, t, 0)),
            pl.BlockSpec((1, Rn, D), lambda b, t: (b, t, 0)),
        ],
        out_shape=[
            jax.ShapeDtypeStruct((B, N, D), jnp.float32),
            jax.ShapeDtypeStruct((B, N, D), jnp.float32),
        ],
    )(feat, xyz, W1)


def _sc_walk_gather(knn2, rand_steps, pflat, B, N, D):
    info = plsc.get_sparse_core_info()
    NC, NS = info.num_cores, info.num_subcores
    NW = NC * NS
    C = N // NW
    mesh = plsc.VectorSubcoreMesh(core_axis_name="c", subcore_axis_name="s")

    @functools.partial(
        pl.kernel,
        mesh=mesh,
        compiler_params=pltpu.CompilerParams(
            needs_layout_passes=False, use_tc_tiling_on_sc=False),
        out_type=jax.ShapeDtypeStruct((B * L * N, D), jnp.float32),
        scratch_types=[
            pltpu.VMEM((C, K_NN), jnp.int32),
            pltpu.VMEM((C,), jnp.int32),
            pltpu.VMEM(((L - 1) * C,), jnp.int32),
            pltpu.VMEM((C, D), jnp.float32),
            pltpu.SemaphoreType.DMA,
            pltpu.SemaphoreType.DMA,
        ],
    )
    def k(knn_hbm, rand_hbm, p_hbm, out_hbm, krows_v, cur_v, rnd_v,
          rows_v, semp, semk):
        wid = lax.axis_index("s") * NC + lax.axis_index("c")
        base = wid * C
        for b in range(B):
            for s in range(L - 1):
                pltpu.sync_copy(rand_hbm.at[s, b, pl.ds(base, C)],
                                rnd_v.at[pl.ds(s * C, C)])
            for j in range(C // 16):
                cur_v[pl.ds(j * 16, 16)] = (
                    lax.iota(jnp.int32, 16) + (b * N + base + j * 16))
            knn_dma = pltpu.async_copy(knn_hbm.at[cur_v], krows_v, semk)
            pltpu.sync_copy(p_hbm.at[pl.ds(b * N + base, C)], rows_v)
            pltpu.sync_copy(rows_v, out_hbm.at[pl.ds(b * L * N + base, C)])
            for s in range(L - 1):
                knn_dma.wait()
                for j in range(C // 16):
                    loc = lax.iota(jnp.int32, 16) + (j * 16)
                    rd = rnd_v[pl.ds(s * C + j * 16, 16)]
                    nxt = plsc.load_gather(krows_v, [loc, rd])
                    cur_v[pl.ds(j * 16, 16)] = nxt + (b * N)
                p_dma = pltpu.async_copy(p_hbm.at[cur_v], rows_v, semp)
                if s < L - 2:
                    knn_dma = pltpu.async_copy(
                        knn_hbm.at[cur_v], krows_v, semk)
                p_dma.wait()
                pltpu.sync_copy(
                    rows_v,
                    out_hbm.at[pl.ds((b * L + s + 1) * N + base, C)])

    return k(knn2, rand_steps, pflat)


def _mvinv(stats_row, count, g, bparm):
    m = stats_row[0:1, :] / count
    v = stats_row[1:2, :] / count - m * m
    inv = lax.rsqrt(v + EPS)
    scale = inv * g.reshape(1, -1)
    shift = bparm.reshape(1, -1) - m * scale
    return scale, shift


def _conv2_body(g_ref, hx_ref, w2_ref, g1_ref, b1_ref,
                m2_ref, s2_ref, s1_scr, *, count1):
    p = pl.program_id(0)
    b = pl.program_id(1)
    t = pl.program_id(2)
    li = pl.program_id(3)
    first = (b == 0) & (t == 0) & (li == 0)

    h = g_ref[0, 0] - hx_ref[0, 0]

    @pl.when((p == 0) & first)
    def _():
        s1_scr[...] = jnp.zeros_like(s1_scr)
        s2_ref[...] = jnp.zeros_like(s2_ref)

    @pl.when(p == 0)
    def _():
        s1_scr[0:1, :] += jnp.sum(h, axis=0, keepdims=True)
        s1_scr[1:2, :] += jnp.sum(h * h, axis=0, keepdims=True)

    @pl.when(p == 1)
    def _():
        scale, shift = _mvinv(s1_scr, count1, g1_ref[...], b1_ref[...])
        a = jnp.maximum(h * scale + shift, 0.0)
        h2 = lax.dot_general(a, w2_ref[...], (((1,), (1,)), ((), ())),
                             preferred_element_type=jnp.float32)
        s2_ref[0:1, :] += jnp.sum(h2, axis=0, keepdims=True)
        s2_ref[1:2, :] += jnp.sum(h2 * h2, axis=0, keepdims=True)

        @pl.when(li == 0)
        def _():
            m2_ref[0] = h2

        @pl.when(li > 0)
        def _():
            m2_ref[0] = jnp.maximum(m2_ref[0], h2)


def _conv2(gall, hxt, W2, g1, b1, B, N, D, Rn=2048):
    count1 = float(B * N * L)
    return pl.pallas_call(
        functools.partial(_conv2_body, count1=count1),
        grid=(2, B, N // Rn, L),
        in_specs=[
            pl.BlockSpec((1, 1, Rn, D), lambda p, b, t, li: (b, li, t, 0)),
            pl.BlockSpec((1, 1, Rn, D), lambda p, b, t, li: (b, 0, t, 0)),
            pl.BlockSpec((D, D), lambda p, b, t, li: (0, 0)),
            pl.BlockSpec((D,), lambda p, b, t, li: (0,)),
            pl.BlockSpec((D,), lambda p, b, t, li: (0,)),
        ],
        out_specs=[
            pl.BlockSpec((1, Rn, D), lambda p, b, t, li: (b * p, t * p, 0)),
            pl.BlockSpec((2, D), lambda p, b, t, li: (0, 0)),
        ],
        out_shape=[
            jax.ShapeDtypeStruct((B, N, D), jnp.float32),
            jax.ShapeDtypeStruct((2, D), jnp.float32),
        ],
        scratch_shapes=[pltpu.VMEM((2, D), jnp.float32)],
    )(gall, hxt, W2, g1, b1)


def _resid_body(m2_ref, featT_ref, s2_ref, g2_ref, b2_ref, u_ref, su_ref,
                *, count2):
    b = pl.program_id(0)
    t = pl.program_id(1)

    @pl.when((b == 0) & (t == 0))
    def _():
        su_ref[...] = jnp.zeros_like(su_ref)

    scale, shift = _mvinv(s2_ref, count2, g2_ref[...], b2_ref[...])
    agg = jnp.maximum(m2_ref[0] * scale + shift, 0.0)
    u = featT_ref[0] + agg
    u_ref[0] = u
    su_ref[0:1, :] += jnp.sum(u, axis=0, keepdims=True)
    su_ref[1:2, :] += jnp.sum(u * u, axis=0, keepdims=True)


def _resid(m2, featT, stats2, g2, b2, B, N, D, Rn=2048):
    count2 = float(B * N * L)
    return pl.pallas_call(
        functools.partial(_resid_body, count2=count2),
        grid=(B, N // Rn),
        in_specs=[
            pl.BlockSpec((1, Rn, D), lambda b, t: (b, t, 0)),
            pl.BlockSpec((1, Rn, D), lambda b, t: (b, t, 0)),
            pl.BlockSpec((2, D), lambda b, t: (0, 0)),
            pl.BlockSpec((D,), lambda b, t: (0,)),
            pl.BlockSpec((D,), lambda b, t: (0,)),
        ],
        out_specs=[
            pl.BlockSpec((1, Rn, D), lambda b, t: (b, t, 0)),
            pl.BlockSpec((2, D), lambda b, t: (0, 0)),
        ],
        out_shape=[
            jax.ShapeDtypeStruct((B, N, D), jnp.float32),
            jax.ShapeDtypeStruct((2, D), jnp.float32),
        ],
    )(m2, featT, stats2, g2, b2)


def _mlp1_body(u_ref, su_ref, w3_ref, gn_ref, bn_ref, ff_ref, s3_ref,
               *, countu):
    b = pl.program_id(0)
    t = pl.program_id(1)

    @pl.when((b == 0) & (t == 0))
    def _():
        s3_ref[...] = jnp.zeros_like(s3_ref)

    scale, shift = _mvinv(su_ref, countu, gn_ref[...], bn_ref[...])
    f = u_ref[0] * scale + shift
    ff = lax.dot_general(f, w3_ref[...], (((1,), (1,)), ((), ())),
                         preferred_element_type=jnp.float32)
    ff_ref[0] = ff
    s3_ref[0:1, :] += jnp.sum(ff, axis=0, keepdims=True)
    s3_ref[1:2, :] += jnp.sum(ff * ff, axis=0, keepdims=True)


def _mlp1(u, statsu, W3, gn, bn_, B, N, D, Rn=2048):
    countu = float(B * N)
    return pl.pallas_call(
        functools.partial(_mlp1_body, countu=countu),
        grid=(B, N // Rn),
        in_specs=[
            pl.BlockSpec((1, Rn, D), lambda b, t: (b, t, 0)),
            pl.BlockSpec((2, D), lambda b, t: (0, 0)),
            pl.BlockSpec((2 * D, D), lambda b, t: (0, 0)),
            pl.BlockSpec((D,), lambda b, t: (0,)),
            pl.BlockSpec((D,), lambda b, t: (0,)),
        ],
        out_specs=[
            pl.BlockSpec((1, Rn, 2 * D), lambda b, t: (b, t, 0)),
            pl.BlockSpec((2, 2 * D), lambda b, t: (0, 0)),
        ],
        out_shape=[
            jax.ShapeDtypeStruct((B, N, 2 * D), jnp.float32),
            jax.ShapeDtypeStruct((2, 2 * D), jnp.float32),
        ],
    )(u, statsu, W3, gn, bn_)


def _mlp2_body(ff_ref, s3_ref, u_ref, su_ref, w4_ref, g3_ref, b3_ref,
               gn_ref, bn_ref, v_ref, sv_ref, *, countu):
    b = pl.program_id(0)
    t = pl.program_id(1)

    @pl.when((b == 0) & (t == 0))
    def _():
        sv_ref[...] = jnp.zeros_like(sv_ref)

    scale3, shift3 = _mvinv(s3_ref, countu, g3_ref[...], b3_ref[...])
    a = jnp.maximum(ff_ref[0] * scale3 + shift3, 0.0)
    ff2 = lax.dot_general(a, w4_ref[...], (((1,), (1,)), ((), ())),
                          preferred_element_type=jnp.float32)
    scaleu, shiftu = _mvinv(su_ref, countu, gn_ref[...], bn_ref[...])
    f = u_ref[0] * scaleu + shiftu
    v = f + ff2
    v_ref[0] = v
    sv_ref[0:1, :] += jnp.sum(v, axis=0, keepdims=True)
    sv_ref[1:2, :] += jnp.sum(v * v, axis=0, keepdims=True)


def _mlp2(ff, stats3, u, statsu, W4, g3, b3, gn, bn_, B, N, D, Rn=2048):
    countu = float(B * N)
    return pl.pallas_call(
        functools.partial(_mlp2_body, countu=countu),
        grid=(B, N // Rn),
        in_specs=[
            pl.BlockSpec((1, Rn, 2 * D), lambda b, t: (b, t, 0)),
            pl.BlockSpec((2, 2 * D), lambda b, t: (0, 0)),
            pl.BlockSpec((1, Rn, D), lambda b, t: (b, t, 0)),
            pl.BlockSpec((2, D), lambda b, t: (0, 0)),
            pl.BlockSpec((D, 2 * D), lambda b, t: (0, 0)),
            pl.BlockSpec((2 * D,), lambda b, t: (0,)),
            pl.BlockSpec((2 * D,), lambda b, t: (0,)),
            pl.BlockSpec((D,), lambda b, t: (0,)),
            pl.BlockSpec((D,), lambda b, t: (0,)),
        ],
        out_specs=[
            pl.BlockSpec((1, Rn, D), lambda b, t: (b, t, 0)),
            pl.BlockSpec((2, D), lambda b, t: (0, 0)),
        ],
        out_shape=[
            jax.ShapeDtypeStruct((B, N, D), jnp.float32),
            jax.ShapeDtypeStruct((2, D), jnp.float32),
        ],
    )(ff, stats3, u, statsu, W4, g3, b3, gn, bn_)


def _final_body(v_ref, sv_ref, gn2_ref, bn2_ref, o_ref, *, countu):
    scale, shift = _mvinv(sv_ref, countu, gn2_ref[...], bn2_ref[...])
    o_ref[0] = v_ref[0] * scale + shift


def _final(v, statsv, gn2, bn2, B, N, D, Rn=2048):
    countu = float(B * N)
    return pl.pallas_call(
        functools.partial(_final_body, countu=countu),
        grid=(B, N // Rn),
        in_specs=[
            pl.BlockSpec((1, Rn, D), lambda b, t: (b, t, 0)),
            pl.BlockSpec((2, D), lambda b, t: (0, 0)),
            pl.BlockSpec((D,), lambda b, t: (0,)),
            pl.BlockSpec((D,), lambda b, t: (0,)),
        ],
        out_specs=pl.BlockSpec((1, Rn, D), lambda b, t: (b, t, 0)),
        out_shape=jax.ShapeDtypeStruct((B, N, D), jnp.float32),
    )(v, statsv, gn2, bn2)


def kernel(xyz, feat, rand_steps, W1, g1, b1, W2, g2, b2, gn, bn_, W3, g3,
           b3, W4, gn2, bn2):
    B, N, _ = xyz.shape
    D = feat.shape[1]

    knn = _knn_topk(xyz, B, N)
    p, hxt = _projections(feat, xyz, W1, B, N, D)

    gall = _sc_walk_gather(
        knn.reshape(B * N, K_NN), rand_steps, p.reshape(B * N, D), B, N, D)
    gall = gall.reshape(B, L, N, D)
    hxt4 = hxt.reshape(B, 1, N, D)

    m2, stats2 = _conv2(gall, hxt4, W2, g1, b1, B, N, D)

    featT = jnp.transpose(feat, (0, 2, 1))
    u, statsu = _resid(m2, featT, stats2, g2, b2, B, N, D)
    ff, stats3 = _mlp1(u, statsu, W3, gn, bn_, B, N, D)
    v, statsv = _mlp2(ff, stats3, u, statsu, W4, g3, b3, gn, bn_, B, N, D)
    out = _final(v, statsv, gn2, bn2, B, N, D)
    return jnp.transpose(out, (0, 2, 1))


# speedup vs baseline: 1.0123x; 1.0082x over previous
import functools

import jax
import jax.numpy as jnp
from jax import lax
from jax.experimental import pallas as pl
from jax.experimental.pallas import tpu as pltpu
from jax.experimental.pallas import tpu_sc as plsc

K_NN = 16
L = 4
EPS = 1e-5


def _topk_body(xyzr_ref, xyzc_ref, knn_ref):
    xr = xyzr_ref[0, :, 0:1]
    yr = xyzr_ref[0, :, 1:2]
    zr = xyzr_ref[0, :, 2:3]
    xc = xyzc_ref[0, 0:1, :]
    yc = xyzc_ref[0, 1:2, :]
    zc = xyzc_ref[0, 2:3, :]
    sqr = xr * xr + yr * yr + zr * zr
    sqc = xc * xc + yc * yc + zc * zc
    bf = lambda t: t.astype(jnp.bfloat16).astype(jnp.float32)
    dot = bf(xr) * bf(xc) + bf(yr) * bf(yc) + bf(zr) * bf(zc)
    arr = (sqr + sqc) - 2.0 * dot
    R = arr.shape[0]
    Nn = arr.shape[1]
    fcol = lax.broadcasted_iota(jnp.int32, (R, Nn), 1).astype(jnp.float32)
    big_f = jnp.float32(1e9)
    inf = jnp.float32(jnp.inf)
    outs = []
    for _ in range(K_NN):
        m = jnp.min(arr, axis=1, keepdims=True)
        cand = jnp.where(arr == m, fcol, big_f)
        amin = jnp.min(cand, axis=1, keepdims=True)
        outs.append(amin)
        arr = jnp.where(fcol == amin, inf, arr)
    knn_ref[0, :, :] = jnp.concatenate(outs, axis=1).astype(jnp.int32)


def _knn_topk(xyz, B, N):
    R = 512
    xyzT = jnp.transpose(xyz, (0, 2, 1))
    return pl.pallas_call(
        _topk_body,
        grid=(B, N // R),
        in_specs=[
            pl.BlockSpec((1, R, 3), lambda b, t: (b, t, 0)),
            pl.BlockSpec((1, 3, N), lambda b, t: (b, 0, 0)),
        ],
        out_specs=pl.BlockSpec((1, R, K_NN), lambda b, t: (b, t, 0)),
        out_shape=jax.ShapeDtypeStruct((B, N, K_NN), jnp.int32),
    )(xyz, xyzT)


def _proj_body(feat_ref, xyz_ref, w1_ref, p_ref, hx_ref):
    w1f = w1_ref[:, 0:128]
    fb = feat_ref[0]
    pf = lax.dot_general(fb, w1f, (((0,), (1,)), ((), ())),
                         preferred_element_type=jnp.float32)
    xb = xyz_ref[0]
    hx = (xb[:, 0:1] * w1_ref[:, 128:129].T
          + xb[:, 1:2] * w1_ref[:, 129:130].T
          + xb[:, 2:3] * w1_ref[:, 130:131].T)
    hx_ref[0] = hx
    p_ref[0] = pf + hx


def _projections(feat, xyz, W1, B, N, D):
    Rn = 1024
    return pl.pallas_call(
        _proj_body,
        grid=(B, N // Rn),
        in_specs=[
            pl.BlockSpec((1, D, Rn), lambda b, t: (b, 0, t)),
            pl.BlockSpec((1, Rn, 3), lambda b, t: (b, t, 0)),
            pl.BlockSpec((D, D + 3), lambda b, t: (0, 0)),
        ],
        out_specs=[
            pl.BlockSpec((1, Rn, D), lambda b, t: (b, t, 0)),
            pl.BlockSpec((1, Rn, D), lambda b, t: (b, t, 0)),
        ],
        out_shape=[
            jax.ShapeDtypeStruct((B, N, D), jnp.float32),
            jax.ShapeDtypeStruct((B, N, D), jnp.float32),
        ],
    )(feat, xyz, W1)


def _sc_walk_gather(knn2, rand_steps, pflat, B, N, D):
    info = plsc.get_sparse_core_info()
    NC, NS = info.num_cores, info.num_subcores
    NW = NC * NS
    C = N // NW
    mesh = plsc.VectorSubcoreMesh(core_axis_name="c", subcore_axis_name="s")

    @functools.partial(
        pl.kernel,
        mesh=mesh,
        compiler_params=pltpu.CompilerParams(
            needs_layout_passes=False, use_tc_tiling_on_sc=False),
        out_type=jax.ShapeDtypeStruct((B * L * N, D), jnp.float32),
        scratch_types=[
            pltpu.VMEM((C, K_NN), jnp.int32),
            pltpu.VMEM((C,), jnp.int32),
            pltpu.VMEM(((L - 1) * C,), jnp.int32),
            pltpu.VMEM((C, D), jnp.float32),
            pltpu.SemaphoreType.DMA,
            pltpu.SemaphoreType.DMA,
        ],
    )
    def k(knn_hbm, rand_hbm, p_hbm, out_hbm, krows_v, cur_v, rnd_v,
          rows_v, semp, semk):
        wid = lax.axis_index("s") * NC + lax.axis_index("c")
        base = wid * C
        for b in range(B):
            for s in range(L - 1):
                pltpu.sync_copy(rand_hbm.at[s, b, pl.ds(base, C)],
                                rnd_v.at[pl.ds(s * C, C)])
            for j in range(C // 16):
                cur_v[pl.ds(j * 16, 16)] = (
                    lax.iota(jnp.int32, 16) + (b * N + base + j * 16))
            knn_dma = pltpu.async_copy(knn_hbm.at[cur_v], krows_v, semk)
            pltpu.sync_copy(p_hbm.at[pl.ds(b * N + base, C)], rows_v)
            pltpu.sync_copy(rows_v, out_hbm.at[pl.ds(b * L * N + base, C)])
            for s in range(L - 1):
                knn_dma.wait()
                for j in range(C // 16):
                    loc = lax.iota(jnp.int32, 16) + (j * 16)
                    rd = rnd_v[pl.ds(s * C + j * 16, 16)]
                    nxt = plsc.load_gather(krows_v, [loc, rd])
                    cur_v[pl.ds(j * 16, 16)] = nxt + (b * N)
                p_dma = pltpu.async_copy(p_hbm.at[cur_v], rows_v, semp)
                if s < L - 2:
                    knn_dma = pltpu.async_copy(
                        knn_hbm.at[cur_v], krows_v, semk)
                p_dma.wait()
                pltpu.sync_copy(
                    rows_v,
                    out_hbm.at[pl.ds((b * L + s + 1) * N + base, C)])

    return k(knn2, rand_steps, pflat)


def _mvinv(stats_row, count, g, bparm):
    m = stats_row[0:1, :] / count
    v = stats_row[1:2, :] / count - m * m
    inv = lax.rsqrt(v + EPS)
    scale = inv * g.reshape(1, -1)
    shift = bparm.reshape(1, -1) - m * scale
    return scale, shift


def _conv2_body(g_ref, hx_ref, w2_ref, g1_ref, b1_ref,
                m2_ref, s2_ref, s1_scr, *, count1):
    p = pl.program_id(0)
    b = pl.program_id(1)
    t = pl.program_id(2)
    li = pl.program_id(3)
    first = (b == 0) & (t == 0) & (li == 0)

    h = g_ref[0, 0] - hx_ref[0, 0]

    @pl.when((p == 0) & first)
    def _():
        s1_scr[...] = jnp.zeros_like(s1_scr)
        s2_ref[...] = jnp.zeros_like(s2_ref)

    @pl.when(p == 0)
    def _():
        s1_scr[0:1, :] += jnp.sum(h, axis=0, keepdims=True)
        s1_scr[1:2, :] += jnp.sum(h * h, axis=0, keepdims=True)

    @pl.when(p == 1)
    def _():
        scale, shift = _mvinv(s1_scr, count1, g1_ref[...], b1_ref[...])
        a = jnp.maximum(h * scale + shift, 0.0)
        h2 = lax.dot_general(a, w2_ref[...], (((1,), (1,)), ((), ())),
                             preferred_element_type=jnp.float32)
        s2_ref[0:1, :] += jnp.sum(h2, axis=0, keepdims=True)
        s2_ref[1:2, :] += jnp.sum(h2 * h2, axis=0, keepdims=True)

        @pl.when(li == 0)
        def _():
            m2_ref[0] = h2

        @pl.when(li > 0)
        def _():
            m2_ref[0] = jnp.maximum(m2_ref[0], h2)


def _conv2(gall, hxt, W2, g1, b1, B, N, D, Rn=2048):
    count1 = float(B * N * L)
    return pl.pallas_call(
        functools.partial(_conv2_body, count1=count1),
        grid=(2, B, N // Rn, L),
        in_specs=[
            pl.BlockSpec((1, 1, Rn, D), lambda p, b, t, li: (b, li, t, 0)),
            pl.BlockSpec((1, 1, Rn, D), lambda p, b, t, li: (b, 0, t, 0)),
            pl.BlockSpec((D, D), lambda p, b, t, li: (0, 0)),
            pl.BlockSpec((D,), lambda p, b, t, li: (0,)),
            pl.BlockSpec((D,), lambda p, b, t, li: (0,)),
        ],
        out_specs=[
            pl.BlockSpec((1, Rn, D), lambda p, b, t, li: (b * p, t * p, 0)),
            pl.BlockSpec((2, D), lambda p, b, t, li: (0, 0)),
        ],
        out_shape=[
            jax.ShapeDtypeStruct((B, N, D), jnp.float32),
            jax.ShapeDtypeStruct((2, D), jnp.float32),
        ],
        scratch_shapes=[pltpu.VMEM((2, D), jnp.float32)],
    )(gall, hxt, W2, g1, b1)


def _resid_body(m2_ref, feat_ref, s2_ref, g2_ref, b2_ref, u_ref, su_ref,
                *, count2):
    b = pl.program_id(0)
    t = pl.program_id(1)

    @pl.when((b == 0) & (t == 0))
    def _():
        su_ref[...] = jnp.zeros_like(su_ref)

    scale, shift = _mvinv(s2_ref, count2, g2_ref[...], b2_ref[...])
    agg = jnp.maximum(m2_ref[0] * scale + shift, 0.0)
    u = jnp.transpose(feat_ref[0], (1, 0)) + agg
    u_ref[0] = u
    su_ref[0:1, :] += jnp.sum(u, axis=0, keepdims=True)
    su_ref[1:2, :] += jnp.sum(u * u, axis=0, keepdims=True)


def _resid(m2, feat, stats2, g2, b2, B, N, D, Rn=2048):
    count2 = float(B * N * L)
    return pl.pallas_call(
        functools.partial(_resid_body, count2=count2),
        grid=(B, N // Rn),
        in_specs=[
            pl.BlockSpec((1, Rn, D), lambda b, t: (b, t, 0)),
            pl.BlockSpec((1, D, Rn), lambda b, t: (b, 0, t)),
            pl.BlockSpec((2, D), lambda b, t: (0, 0)),
            pl.BlockSpec((D,), lambda b, t: (0,)),
            pl.BlockSpec((D,), lambda b, t: (0,)),
        ],
        out_specs=[
            pl.BlockSpec((1, Rn, D), lambda b, t: (b, t, 0)),
            pl.BlockSpec((2, D), lambda b, t: (0, 0)),
        ],
        out_shape=[
            jax.ShapeDtypeStruct((B, N, D), jnp.float32),
            jax.ShapeDtypeStruct((2, D), jnp.float32),
        ],
    )(m2, feat, stats2, g2, b2)


def _mlp1_body(u_ref, su_ref, w3_ref, gn_ref, bn_ref, ff_ref, s3_ref,
               *, countu):
    b = pl.program_id(0)
    t = pl.program_id(1)

    @pl.when((b == 0) & (t == 0))
    def _():
        s3_ref[...] = jnp.zeros_like(s3_ref)

    scale, shift = _mvinv(su_ref, countu, gn_ref[...], bn_ref[...])
    f = u_ref[0] * scale + shift
    ff = lax.dot_general(f, w3_ref[...], (((1,), (1,)), ((), ())),
                         preferred_element_type=jnp.float32)
    ff_ref[0] = ff
    s3_ref[0:1, :] += jnp.sum(ff, axis=0, keepdims=True)
    s3_ref[1:2, :] += jnp.sum(ff * ff, axis=0, keepdims=True)


def _mlp1(u, statsu, W3, gn, bn_, B, N, D, Rn=2048):
    countu = float(B * N)
    return pl.pallas_call(
        functools.partial(_mlp1_body, countu=countu),
        grid=(B, N // Rn),
        in_specs=[
            pl.BlockSpec((1, Rn, D), lambda b, t: (b, t, 0)),
            pl.BlockSpec((2, D), lambda b, t: (0, 0)),
            pl.BlockSpec((2 * D, D), lambda b, t: (0, 0)),
            pl.BlockSpec((D,), lambda b, t: (0,)),
            pl.BlockSpec((D,), lambda b, t: (0,)),
        ],
        out_specs=[
            pl.BlockSpec((1, Rn, 2 * D), lambda b, t: (b, t, 0)),
            pl.BlockSpec((2, 2 * D), lambda b, t: (0, 0)),
        ],
        out_shape=[
            jax.ShapeDtypeStruct((B, N, 2 * D), jnp.float32),
            jax.ShapeDtypeStruct((2, 2 * D), jnp.float32),
        ],
    )(u, statsu, W3, gn, bn_)


def _mlp2_body(ff_ref, s3_ref, u_ref, su_ref, w4_ref, g3_ref, b3_ref,
               gn_ref, bn_ref, v_ref, sv_ref, *, countu):
    b = pl.program_id(0)
    t = pl.program_id(1)

    @pl.when((b == 0) & (t == 0))
    def _():
        sv_ref[...] = jnp.zeros_like(sv_ref)

    scale3, shift3 = _mvinv(s3_ref, countu, g3_ref[...], b3_ref[...])
    a = jnp.maximum(ff_ref[0] * scale3 + shift3, 0.0)
    ff2 = lax.dot_general(a, w4_ref[...], (((1,), (1,)), ((), ())),
                          preferred_element_type=jnp.float32)
    scaleu, shiftu = _mvinv(su_ref, countu, gn_ref[...], bn_ref[...])
    f = u_ref[0] * scaleu + shiftu
    v = f + ff2
    v_ref[0] = v
    sv_ref[0:1, :] += jnp.sum(v, axis=0, keepdims=True)
    sv_ref[1:2, :] += jnp.sum(v * v, axis=0, keepdims=True)


def _mlp2(ff, stats3, u, statsu, W4, g3, b3, gn, bn_, B, N, D, Rn=2048):
    countu = float(B * N)
    return pl.pallas_call(
        functools.partial(_mlp2_body, countu=countu),
        grid=(B, N // Rn),
        in_specs=[
            pl.BlockSpec((1, Rn, 2 * D), lambda b, t: (b, t, 0)),
            pl.BlockSpec((2, 2 * D), lambda b, t: (0, 0)),
            pl.BlockSpec((1, Rn, D), lambda b, t: (b, t, 0)),
            pl.BlockSpec((2, D), lambda b, t: (0, 0)),
            pl.BlockSpec((D, 2 * D), lambda b, t: (0, 0)),
            pl.BlockSpec((2 * D,), lambda b, t: (0,)),
            pl.BlockSpec((2 * D,), lambda b, t: (0,)),
            pl.BlockSpec((D,), lambda b, t: (0,)),
            pl.BlockSpec((D,), lambda b, t: (0,)),
        ],
        out_specs=[
            pl.BlockSpec((1, Rn, D), lambda b, t: (b, t, 0)),
            pl.BlockSpec((2, D), lambda b, t: (0, 0)),
        ],
        out_shape=[
            jax.ShapeDtypeStruct((B, N, D), jnp.float32),
            jax.ShapeDtypeStruct((2, D), jnp.float32),
        ],
    )(ff, stats3, u, statsu, W4, g3, b3, gn, bn_)


def _final_body(v_ref, sv_ref, gn2_ref, bn2_ref, o_ref, *, countu):
    scale, shift = _mvinv(sv_ref, countu, gn2_ref[...], bn2_ref[...])
    o_ref[0] = jnp.transpose(v_ref[0] * scale + shift, (1, 0))


def _final(v, statsv, gn2, bn2, B, N, D, Rn=2048):
    countu = float(B * N)
    return pl.pallas_call(
        functools.partial(_final_body, countu=countu),
        grid=(B, N // Rn),
        in_specs=[
            pl.BlockSpec((1, Rn, D), lambda b, t: (b, t, 0)),
            pl.BlockSpec((2, D), lambda b, t: (0, 0)),
            pl.BlockSpec((D,), lambda b, t: (0,)),
            pl.BlockSpec((D,), lambda b, t: (0,)),
        ],
        out_specs=pl.BlockSpec((1, D, Rn), lambda b, t: (b, 0, t)),
        out_shape=jax.ShapeDtypeStruct((B, D, N), jnp.float32),
    )(v, statsv, gn2, bn2)


def kernel(xyz, feat, rand_steps, W1, g1, b1, W2, g2, b2, gn, bn_, W3, g3,
           b3, W4, gn2, bn2):
    B, N, _ = xyz.shape
    D = feat.shape[1]

    knn = _knn_topk(xyz, B, N)
    p, hxt = _projections(feat, xyz, W1, B, N, D)

    gall = _sc_walk_gather(
        knn.reshape(B * N, K_NN), rand_steps, p.reshape(B * N, D), B, N, D)
    gall = gall.reshape(B, L, N, D)
    hxt4 = hxt.reshape(B, 1, N, D)

    m2, stats2 = _conv2(gall, hxt4, W2, g1, b1, B, N, D)

    u, statsu = _resid(m2, feat, stats2, g2, b2, B, N, D)
    ff, stats3 = _mlp1(u, statsu, W3, gn, bn_, B, N, D)
    v, statsv = _mlp2(ff, stats3, u, statsu, W4, g3, b3, gn, bn_, B, N, D)
    return _final(v, statsv, gn2, bn2, B, N, D)
